# baseline (speedup 1.0000x reference)
import functools

import jax
import jax.numpy as jnp
from jax import lax
from jax.experimental import pallas as pl
from jax.experimental.pallas import tpu as pltpu
from jax.experimental.pallas import tpu_sc as plsc

BATCH = 16384
EMBED_DIM = 64
VROW = 128

_INFO = plsc.get_sparse_core_info()
_NC = _INFO.num_cores
_NS = _INFO.num_subcores
_L = _INFO.num_lanes
_NW = _NC * _NS
_BPW = BATCH // _NW
_CHUNK = 128
_NCHUNK = _BPW // _CHUNK
_SB = 2
_CPS = _NCHUNK // _SB
_RPS = _BPW // _SB


@functools.partial(
    pl.kernel,
    mesh=plsc.VectorSubcoreMesh(core_axis_name="c", subcore_axis_name="s"),
    compiler_params=pltpu.CompilerParams(
        needs_layout_passes=False, use_tc_tiling_on_sc=True),
    out_type=jax.ShapeDtypeStruct((BATCH,), jnp.float32),
    scratch_types=[
        pltpu.VMEM((_NCHUNK, _CHUNK), jnp.int32),
        pltpu.VMEM((_NCHUNK, _CHUNK), jnp.int32),
        pltpu.VMEM((_BPW,), jnp.int32),
        pltpu.VMEM((_RPS, VROW), jnp.float32),
        pltpu.VMEM((_RPS, VROW), jnp.float32),
        pltpu.VMEM((_BPW,), jnp.float32),
        pltpu.SemaphoreType.DMA,
    ],
)
def _sc_dot_kernel(uids_hbm, mids_hbm, utab_hbm, mtab_hbm, out_hbm,
                   uidx_v, midx_v, cbase_v, urows_v, mrows_v, out_v, sem):
    wid = lax.axis_index("s") * _NC + lax.axis_index("c")
    base = wid * _BPW

    pltpu.sync_copy(uids_hbm.at[pl.ds(wid * _NCHUNK, _NCHUNK)], uidx_v)
    pltpu.sync_copy(mids_hbm.at[pl.ds(wid * _NCHUNK, _NCHUNK)], midx_v)

    for j in range(_NCHUNK):
        for g in range(_CHUNK // _L):
            sl = pl.ds(g * _L, _L)
            u = uidx_v[j, sl]
            m = midx_v[j, sl]
            cbase_v[pl.ds(j * _CHUNK + g * _L, _L)] = (
                (u & 1) * ((VROW // 2) * (2 ** 16)) + (m & 1) * (VROW // 2))
            uidx_v[j, sl] = lax.shift_right_logical(u, 1)
            midx_v[j, sl] = lax.shift_right_logical(m, 1)

    for sb in range(_SB):
        copies = []
        for j in range(_CPS):
            dst_u = urows_v.at[pl.ds(j * _CHUNK, _CHUNK)]
            dst_m = mrows_v.at[pl.ds(j * _CHUNK, _CHUNK)]
            idx = sb * _CPS + j
            copies.append(
                pltpu.async_copy(utab_hbm.at[uidx_v.at[idx]], dst_u, sem))
            copies.append(
                pltpu.async_copy(mtab_hbm.at[midx_v.at[idx]], dst_m, sem))
        for c in copies:
            c.wait()

        def group_body(g, carry):
            rows = g * _L + lax.iota(jnp.int32, _L)
            cb = cbase_v[pl.ds(sb * _RPS + g * _L, _L)]
            ucb = lax.shift_right_logical(cb, 16)
            mcb = cb & (VROW - 1)
            acc = jnp.zeros((_L,), jnp.float32)
            for j in range(EMBED_DIM):
                u = plsc.load_gather(urows_v, [rows, ucb + j])
                m = plsc.load_gather(mrows_v, [rows, mcb + j])
                acc = acc + u * m
            out_v[pl.ds(sb * _RPS + g * _L, _L)] = acc
            return carry

        lax.fori_loop(0, _RPS // _L, group_body, 0)

    pltpu.sync_copy(out_v, out_hbm.at[pl.ds(base, _BPW)])


def kernel(user_ids, movie_ids, user_table, movie_table):
    uids = user_ids.astype(jnp.int32).reshape(_NW * _NCHUNK, _CHUNK)
    mids = movie_ids.astype(jnp.int32).reshape(_NW * _NCHUNK, _CHUNK)
    utab = user_table.reshape(-1, VROW)
    mtab = movie_table.reshape(-1, VROW)
    return _sc_dot_kernel(uids, mids, utab, mtab)


# speedup vs baseline: 1.5846x; 1.5846x over previous
import functools

import jax
import jax.numpy as jnp
from jax import lax
from jax.experimental import pallas as pl
from jax.experimental.pallas import tpu as pltpu
from jax.experimental.pallas import tpu_sc as plsc

BATCH = 16384
EMBED_DIM = 64
TILE = 8

_INFO = plsc.get_sparse_core_info()
_NC = _INFO.num_cores
_NS = _INFO.num_subcores
_L = _INFO.num_lanes
_NW = _NC * _NS
_BPW = BATCH // _NW
_CHUNK = 16
_NCHUNK = _BPW // _CHUNK


@functools.partial(
    pl.kernel,
    mesh=plsc.VectorSubcoreMesh(core_axis_name="c", subcore_axis_name="s"),
    compiler_params=pltpu.CompilerParams(needs_layout_passes=False),
    out_type=jax.ShapeDtypeStruct((BATCH,), jnp.float32),
    scratch_types=[
        pltpu.VMEM((_BPW,), jnp.int32),
        pltpu.VMEM((_BPW,), jnp.int32),
        pltpu.VMEM((2, _CHUNK, TILE, EMBED_DIM), jnp.float32),
        pltpu.VMEM((2, _CHUNK, TILE, EMBED_DIM), jnp.float32),
        pltpu.VMEM((_BPW,), jnp.float32),
        pltpu.SemaphoreType.DMA,
        pltpu.SemaphoreType.DMA,
    ],
)
def _sc_dot_kernel(uids_hbm, mids_hbm, utab_hbm, mtab_hbm, out_hbm,
                   uidx_v, midx_v, utile_v, mtile_v, out_v, sem0, sem1):
    wid = lax.axis_index("s") * _NC + lax.axis_index("c")
    base = wid * _BPW
    sems = (sem0, sem1)

    pltpu.sync_copy(uids_hbm.at[pl.ds(base, _BPW)], uidx_v)
    pltpu.sync_copy(mids_hbm.at[pl.ds(base, _BPW)], midx_v)

    def fire(c, buf):
        off = pl.multiple_of(c * _CHUNK, TILE)
        u16 = uidx_v[pl.ds(off, _L)]
        m16 = midx_v[pl.ds(off, _L)]
        for j in range(_L):
            ub = pl.multiple_of(u16[j] & ~(TILE - 1), TILE)
            mb = pl.multiple_of(m16[j] & ~(TILE - 1), TILE)
            pltpu.async_copy(
                utab_hbm.at[pl.ds(ub, TILE)], utile_v.at[buf, j], sems[buf])
            pltpu.async_copy(
                mtab_hbm.at[pl.ds(mb, TILE)], mtile_v.at[buf, j], sems[buf])

    def drain(buf):
        for j in range(_L):
            pltpu.make_async_copy(
                utab_hbm.at[pl.ds(0, TILE)], utile_v.at[buf, j],
                sems[buf]).wait()
            pltpu.make_async_copy(
                mtab_hbm.at[pl.ds(0, TILE)], mtile_v.at[buf, j],
                sems[buf]).wait()

    def compute(c, buf):
        off = pl.multiple_of(c * _CHUNK, TILE)
        slots = lax.iota(jnp.int32, _L)
        usub = uidx_v[pl.ds(off, _L)] & (TILE - 1)
        msub = midx_v[pl.ds(off, _L)] & (TILE - 1)
        acc = jnp.zeros((_L,), jnp.float32)
        for k in range(EMBED_DIM):
            cols = jnp.full((_L,), k, jnp.int32)
            u = plsc.load_gather(utile_v.at[buf], [slots, usub, cols])
            m = plsc.load_gather(mtile_v.at[buf], [slots, msub, cols])
            acc = acc + u * m
        out_v[pl.ds(off, _L)] = acc

    fire(0, 0)
    fire(1, 1)

    @pl.loop(0, _NCHUNK - 2, step=2)
    def _ring(c):
        drain(0)
        compute(c, 0)
        fire(c + 2, 0)
        drain(1)
        compute(c + 1, 1)
        fire(c + 3, 1)

    drain(0)
    compute(_NCHUNK - 2, 0)
    drain(1)
    compute(_NCHUNK - 1, 1)

    pltpu.sync_copy(out_v, out_hbm.at[pl.ds(base, _BPW)])


def kernel(user_ids, movie_ids, user_table, movie_table):
    uids = user_ids.astype(jnp.int32)
    mids = movie_ids.astype(jnp.int32)
    return _sc_dot_kernel(uids, mids, user_table, movie_table)


# speedup vs baseline: 1.6449x; 1.0380x over previous
import functools

import jax
import jax.numpy as jnp
from jax import lax
from jax.experimental import pallas as pl
from jax.experimental.pallas import tpu as pltpu
from jax.experimental.pallas import tpu_sc as plsc

BATCH = 16384
EMBED_DIM = 64

_INFO = plsc.get_sparse_core_info()
_NC = _INFO.num_cores
_NS = _INFO.num_subcores
_L = _INFO.num_lanes
_NW = _NC * _NS
_BPW = BATCH // _NW
_CHUNK = 16
_NCHUNK = _BPW // _CHUNK


@functools.partial(
    pl.kernel,
    mesh=plsc.VectorSubcoreMesh(core_axis_name="c", subcore_axis_name="s"),
    compiler_params=pltpu.CompilerParams(needs_layout_passes=False),
    out_type=jax.ShapeDtypeStruct((BATCH,), jnp.float32),
    scratch_types=[
        pltpu.VMEM((_BPW,), jnp.int32),
        pltpu.VMEM((_BPW,), jnp.int32),
        pltpu.VMEM((2, _CHUNK, EMBED_DIM), jnp.float32),
        pltpu.VMEM((2, _CHUNK, EMBED_DIM), jnp.float32),
        pltpu.VMEM((_BPW,), jnp.float32),
        pltpu.SemaphoreType.DMA,
        pltpu.SemaphoreType.DMA,
    ],
)
def _sc_dot_kernel(uids_hbm, mids_hbm, utab_hbm, mtab_hbm, out_hbm,
                   uidx_v, midx_v, urows_v, mrows_v, out_v, sem0, sem1):
    wid = lax.axis_index("s") * _NC + lax.axis_index("c")
    base = wid * _BPW
    sems = (sem0, sem1)

    pltpu.sync_copy(uids_hbm.at[pl.ds(base, _BPW)], uidx_v)
    pltpu.sync_copy(mids_hbm.at[pl.ds(base, _BPW)], midx_v)

    def fire(c, buf):
        off = pl.multiple_of(c * _CHUNK, _L)
        u16 = uidx_v[pl.ds(off, _L)]
        m16 = midx_v[pl.ds(off, _L)]
        for j in range(_L):
            pltpu.async_copy(
                utab_hbm.at[u16[j]], urows_v.at[buf, j], sems[buf])
            pltpu.async_copy(
                mtab_hbm.at[m16[j]], mrows_v.at[buf, j], sems[buf])

    def drain(buf):
        for j in range(_L):
            pltpu.make_async_copy(
                utab_hbm.at[0], urows_v.at[buf, j], sems[buf]).wait()
            pltpu.make_async_copy(
                mtab_hbm.at[0], mrows_v.at[buf, j], sems[buf]).wait()

    def compute(c, buf):
        off = pl.multiple_of(c * _CHUNK, _L)
        slots = lax.iota(jnp.int32, _L)
        acc = jnp.zeros((_L,), jnp.float32)
        for k in range(EMBED_DIM):
            cols = jnp.full((_L,), k, jnp.int32)
            u = plsc.load_gather(urows_v.at[buf], [slots, cols])
            m = plsc.load_gather(mrows_v.at[buf], [slots, cols])
            acc = acc + u * m
        out_v[pl.ds(off, _L)] = acc

    fire(0, 0)
    fire(1, 1)

    @pl.loop(0, _NCHUNK - 2, step=2)
    def _ring(c):
        drain(0)
        compute(c, 0)
        fire(c + 2, 0)
        drain(1)
        compute(c + 1, 1)
        fire(c + 3, 1)

    drain(0)
    compute(_NCHUNK - 2, 0)
    drain(1)
    compute(_NCHUNK - 1, 1)

    pltpu.sync_copy(out_v, out_hbm.at[pl.ds(base, _BPW)])


def kernel(user_ids, movie_ids, user_table, movie_table):
    uids = user_ids.astype(jnp.int32)
    mids = movie_ids.astype(jnp.int32)
    return _sc_dot_kernel(uids, mids, user_table, movie_table)
